# initial kernel scaffold (unmeasured)
import jax
import jax.numpy as jnp
from jax import lax
from jax.experimental import pallas as pl
from jax.experimental.pallas import tpu as pltpu

N_DEV = 16
M_PER = 256
K = 4096
N = 8192
N_PER = 512


def kernel(x, w_mat):
    assert x.shape == (M_PER, K), x.shape
    assert w_mat.shape == (K, N), w_mat.shape

    def body(x_ref, w_hbm, out_ref,
             wbuf, y_ref, amax_buf, qsend, qstage,
             w_sems, asend, arecv, tsend, trecv):
        my = lax.axis_index("i")

        def w_copy(j, slot):
            return pltpu.make_async_copy(
                w_hbm.at[:, pl.ds(j * N_PER, N_PER)],
                wbuf.at[slot],
                w_sems.at[slot],
            )

        w_copy(0, 0).start()
        local_amax = jnp.float32(0.0)
        for j in range(N_DEV):
            slot = j % 2
            if j + 1 < N_DEV:
                w_copy(j + 1, (j + 1) % 2).start()
            w_copy(j, slot).wait()
            yj = jnp.dot(x_ref[...], wbuf[slot],
                         preferred_element_type=jnp.float32)
            y_ref[j] = yj
            local_amax = jnp.maximum(local_amax, jnp.max(jnp.abs(yj)))

        amax_buf[0] = jnp.full((8, 128), local_amax, jnp.float32)

        def amax_rdma(k):
            return pltpu.make_async_remote_copy(
                src_ref=amax_buf.at[0],
                dst_ref=amax_buf.at[k],
                send_sem=asend.at[k],
                recv_sem=arecv.at[k],
                device_id=(lax.rem(my + k, N_DEV),),
                device_id_type=pl.DeviceIdType.MESH,
            )

        for k in range(1, N_DEV):
            amax_rdma(k).start()
        for k in range(1, N_DEV):
            amax_rdma(k).wait_send()
        for k in range(1, N_DEV):
            amax_rdma(k).wait_recv()
        scale = jnp.max(amax_buf[...]) / 448.0

        def tile_rdma(k):
            return pltpu.make_async_remote_copy(
                src_ref=qsend.at[k],
                dst_ref=qstage.at[k],
                send_sem=tsend.at[k],
                recv_sem=trecv.at[k],
                device_id=(lax.rem(my + k, N_DEV),),
                device_id_type=pl.DeviceIdType.MESH,
            )

        for k in range(1, N_DEV):
            dst = lax.rem(my + k, N_DEV)
            qsend[k] = (y_ref[dst] / scale).astype(jnp.float8_e4m3fn)
            tile_rdma(k).start()

        q_own = (y_ref[my] / scale).astype(jnp.float8_e4m3fn)
        out_ref[pl.ds(my * M_PER, M_PER), :] = (
            q_own.astype(jnp.float32) * scale)

        for k in range(1, N_DEV):
            tile_rdma(k).wait_recv()
            src = lax.rem(my - k + N_DEV, N_DEV)
            out_ref[pl.ds(src * M_PER, M_PER), :] = (
                qstage[k].astype(jnp.float32) * scale)
        for k in range(1, N_DEV):
            tile_rdma(k).wait_send()

    return pl.pallas_call(
        body,
        out_shape=jax.ShapeDtypeStruct((N_DEV * M_PER, N_PER), jnp.float32),
        in_specs=[
            pl.BlockSpec(memory_space=pltpu.VMEM),
            pl.BlockSpec(memory_space=pltpu.ANY),
        ],
        out_specs=pl.BlockSpec(memory_space=pltpu.VMEM),
        scratch_shapes=[
            pltpu.VMEM((2, K, N_PER), jnp.float32),
            pltpu.VMEM((N_DEV, M_PER, N_PER), jnp.float32),
            pltpu.VMEM((N_DEV, 8, 128), jnp.float32),
            pltpu.VMEM((N_DEV, M_PER, N_PER), jnp.float8_e4m3fn),
            pltpu.VMEM((N_DEV, M_PER, N_PER), jnp.float8_e4m3fn),
            pltpu.SemaphoreType.DMA((2,)),
            pltpu.SemaphoreType.DMA((N_DEV,)),
            pltpu.SemaphoreType.DMA((N_DEV,)),
            pltpu.SemaphoreType.DMA((N_DEV,)),
            pltpu.SemaphoreType.DMA((N_DEV,)),
        ],
    )(x, w_mat)


# baseline (device time: 82624 ns/iter reference)
import jax
import jax.numpy as jnp
from jax import lax
from jax.experimental import pallas as pl
from jax.experimental.pallas import tpu as pltpu

N_DEV = 16
M_PER = 256
K = 4096
N = 8192
N_PER = 512


def kernel(x, w_mat):
    assert x.shape == (M_PER, K), x.shape
    assert w_mat.shape == (K, N), w_mat.shape

    def body(x_ref, w_hbm, out_ref,
             wbuf, y_ref, amax_buf, qsend, qstage,
             w_sems, asend, arecv, tsend, trecv):
        my = lax.axis_index("i")

        def w_copy(j, slot):
            return pltpu.make_async_copy(
                w_hbm.at[:, pl.ds(j * N_PER, N_PER)],
                wbuf.at[slot],
                w_sems.at[slot],
            )

        w_copy(0, 0).start()
        local_amax = jnp.float32(0.0)
        for j in range(N_DEV):
            slot = j % 2
            if j + 1 < N_DEV:
                w_copy(j + 1, (j + 1) % 2).start()
            w_copy(j, slot).wait()
            yj = jnp.dot(x_ref[...], wbuf[slot],
                         preferred_element_type=jnp.float32)
            y_ref[j] = yj
            local_amax = jnp.maximum(local_amax, jnp.max(jnp.abs(yj)))

        amax_buf[0] = jnp.full((8, 128), local_amax, jnp.float32)

        def amax_rdma(k):
            return pltpu.make_async_remote_copy(
                src_ref=amax_buf.at[0],
                dst_ref=amax_buf.at[k],
                send_sem=asend.at[k],
                recv_sem=arecv.at[k],
                device_id=(lax.rem(my + k, N_DEV),),
                device_id_type=pl.DeviceIdType.MESH,
            )

        for k in range(1, N_DEV):
            amax_rdma(k).start()
        for k in range(1, N_DEV):
            amax_rdma(k).wait_send()
        for k in range(1, N_DEV):
            amax_rdma(k).wait_recv()
        scale = jnp.max(amax_buf[...]) / 448.0

        def tile_rdma(k):
            return pltpu.make_async_remote_copy(
                src_ref=qsend.at[k],
                dst_ref=qstage.at[k],
                send_sem=tsend.at[k],
                recv_sem=trecv.at[k],
                device_id=(lax.rem(my + k, N_DEV),),
                device_id_type=pl.DeviceIdType.MESH,
            )

        for k in range(1, N_DEV):
            dst = lax.rem(my + k, N_DEV)
            qsend[k] = (y_ref[dst] / scale).astype(jnp.float8_e4m3fn)
            tile_rdma(k).start()

        q_own = (y_ref[my] / scale).astype(jnp.float8_e4m3fn)
        out_ref[pl.ds(my * M_PER, M_PER), :] = (
            q_own.astype(jnp.float32) * scale)

        for k in range(1, N_DEV):
            tile_rdma(k).wait_recv()
            src = lax.rem(my - k + N_DEV, N_DEV)
            out_ref[pl.ds(src * M_PER, M_PER), :] = (
                qstage[k].astype(jnp.float32) * scale)
        for k in range(1, N_DEV):
            tile_rdma(k).wait_send()

    return pl.pallas_call(
        body,
        out_shape=jax.ShapeDtypeStruct((N_DEV * M_PER, N_PER), jnp.float32),
        in_specs=[
            pl.BlockSpec(memory_space=pltpu.MemorySpace.VMEM),
            pl.BlockSpec(memory_space=pltpu.MemorySpace.HBM),
        ],
        out_specs=pl.BlockSpec(memory_space=pltpu.MemorySpace.VMEM),
        scratch_shapes=[
            pltpu.MemorySpace.VMEM((2, K, N_PER), jnp.float32),
            pltpu.MemorySpace.VMEM((N_DEV, M_PER, N_PER), jnp.float32),
            pltpu.MemorySpace.VMEM((N_DEV, 8, 128), jnp.float32),
            pltpu.MemorySpace.VMEM((N_DEV, M_PER, N_PER), jnp.float8_e4m3fn),
            pltpu.MemorySpace.VMEM((N_DEV, M_PER, N_PER), jnp.float8_e4m3fn),
            pltpu.SemaphoreType.DMA((2,)),
            pltpu.SemaphoreType.DMA((N_DEV,)),
            pltpu.SemaphoreType.DMA((N_DEV,)),
            pltpu.SemaphoreType.DMA((N_DEV,)),
            pltpu.SemaphoreType.DMA((N_DEV,)),
        ],
    )(x, w_mat)
